# baseline (speedup 1.0000x reference)
import functools
import numpy as np
import jax
import jax.numpy as jnp
from jax.experimental import pallas as pl

N_ATOMS = 10000
N_EDGES = 320000
N_BASIS = 7
N_RADIAL = 5
N_SPECIES = 119
R_MIN = 0.5
R_MAX = 6.0
FEAT_DIM = 360

NA_PAD = 10240
A_BLK = 2560
E_BLK = 512
A2_BLK = 2048

_BETTA = float(N_BASIS) ** 2 / R_MAX ** 2
_RAD_NORM = (2.0 * _BETTA / np.pi) ** 0.25
_SHIFTS = np.asarray(
    R_MIN + (R_MAX - R_MIN) / N_BASIS * np.arange(N_BASIS), dtype=np.float32
)

_I2, _J2 = np.tril_indices(N_RADIAL)
_TRIL3 = [(i, j, k) for i in range(N_RADIAL)
          for j in range(i + 1) for k in range(j + 1)]


def _edge_kernel(coeffs_ref, ri_ref, rj_ref, out_ref):
    dr = ri_ref[...] - rj_ref[...]
    dr2 = jnp.sum(dr * dr, axis=0, keepdims=True)
    dist = jnp.sqrt(dr2 + 1e-12)
    e = dr / dist
    shifts = R_MIN + (R_MAX - R_MIN) / N_BASIS * jax.lax.broadcasted_iota(
        jnp.int32, (N_BASIS, 1), 0).astype(jnp.float32)
    basis = _RAD_NORM * jnp.exp(-_BETTA * (dist - shifts) ** 2)
    coeffs = coeffs_ref[...].reshape(N_RADIAL, N_BASIS, E_BLK)
    radial = jnp.sum(coeffs * basis[None, :, :], axis=1)
    cutoff = jnp.where(
        dist < R_MAX, 0.5 * (jnp.cos(np.pi / R_MAX * dist) + 1.0), 0.0
    )
    out_ref[0:N_RADIAL, :] = radial * cutoff
    out_ref[N_RADIAL:, :] = e


def _segsum_kernel(red_ref, idx_ref, m_ref):
    a = pl.program_id(0)
    e_step = pl.program_id(1)

    @pl.when(e_step == 0)
    def _():
        m_ref[...] = jnp.zeros_like(m_ref)

    radial = red_ref[0:N_RADIAL, :]
    e = red_ref[N_RADIAL:, :]
    ee = (e[:, None, :] * e[None, :, :]).reshape(9, E_BLK)
    eee = (ee[:, None, :] * e[None, :, :]).reshape(27, E_BLK)
    ones = jnp.ones((1, E_BLK), jnp.float32)
    eprod = jnp.concatenate([ones, e, ee, eee], axis=0)
    payload = (radial[:, None, :] * eprod[None, :, :]).reshape(200, E_BLK)

    row = jax.lax.broadcasted_iota(
        jnp.int32, (A_BLK, E_BLK), 0).astype(jnp.float32)
    base = (a * A_BLK).astype(jnp.float32)
    onehot = jnp.where(row + base == idx_ref[...], 1.0, 0.0
                       ).astype(jnp.bfloat16)
    p_hi = payload.astype(jnp.bfloat16)
    p_lo = (payload - p_hi.astype(jnp.float32)).astype(jnp.bfloat16)
    dims = (((1,), (1,)), ((), ()))
    m_ref[...] += (
        jax.lax.dot_general(onehot, p_hi, dims,
                            preferred_element_type=jnp.float32)
        + jax.lax.dot_general(onehot, p_lo, dims,
                              preferred_element_type=jnp.float32))


def _swish(x):
    return x / (1.0 + jnp.exp(-x))


@functools.partial(jax.jit)
def _impl(R, Z, neighbor_idx, radial_emb, W0, b0, W1, b1, W2, b2, scale, shift):
    idx_i = neighbor_idx[0].astype(jnp.int32)
    idx_j = neighbor_idx[1].astype(jnp.int32)
    Zi = Z[idx_i]
    Zj = Z[idx_j]
    riT = R[idx_i].T
    rjT = R[idx_j].T
    coeffsT = radial_emb.reshape(N_SPECIES * N_SPECIES, N_RADIAL * N_BASIS)[
        Zi * N_SPECIES + Zj].T

    red = pl.pallas_call(
        _edge_kernel,
        grid=(N_EDGES // E_BLK,),
        in_specs=[
            pl.BlockSpec((N_RADIAL * N_BASIS, E_BLK), lambda e: (0, e)),
            pl.BlockSpec((3, E_BLK), lambda e: (0, e)),
            pl.BlockSpec((3, E_BLK), lambda e: (0, e)),
        ],
        out_specs=pl.BlockSpec((8, E_BLK), lambda e: (0, e)),
        out_shape=jax.ShapeDtypeStruct((8, N_EDGES), jnp.float32),
    )(coeffsT, riT, rjT)

    idxf = idx_i.astype(jnp.float32)[None, :]
    m = pl.pallas_call(
        _segsum_kernel,
        grid=(NA_PAD // A_BLK, N_EDGES // E_BLK),
        in_specs=[
            pl.BlockSpec((8, E_BLK), lambda a, e: (0, e)),
            pl.BlockSpec((1, E_BLK), lambda a, e: (0, e)),
        ],
        out_specs=pl.BlockSpec((A_BLK, 200), lambda a, e: (a, 0)),
        out_shape=jax.ShapeDtypeStruct((NA_PAD, 200), jnp.float32),
    )(red, idxf)

    mT = m.T
    maskZ = jnp.zeros((1, NA_PAD), jnp.float32).at[0, :N_ATOMS].set(
        (Z > 0).astype(jnp.float32))
    scaleZ = jnp.zeros((1, NA_PAD), jnp.float32).at[0, :N_ATOMS].set(
        scale[Z, 0])
    shiftZ = jnp.zeros((1, NA_PAD), jnp.float32).at[0, :N_ATOMS].set(
        shift[Z, 0])

    def _k2(m_ref, w0_ref, b0_ref, w1_ref, b1_ref, w2_ref, b2_ref,
            scale_ref, shift_ref, mask_ref, out_ref):
        A = A2_BLK
        mm = m_ref[...].reshape(N_RADIAL, 40, A)
        m0 = mm[:, 0, :]
        m1 = mm[:, 1:4, :]
        m2f = mm[:, 4:13, :]
        m2 = m2f.reshape(N_RADIAL, 3, 3, A)
        m3f = mm[:, 13:40, :]
        m3 = m3f.reshape(N_RADIAL, 9, 3, A)

        c1 = jnp.sum(m1[:, None] * m1[None, :], axis=2)
        c2 = jnp.sum(m2f[:, None] * m2f[None, :], axis=2)
        c3 = jnp.sum(m3f[:, None] * m3f[None, :], axis=2)
        t4 = jnp.sum(m2[:, None, :, :, None] * m2[None, :, :, None, :], axis=2)
        c4 = jnp.sum(t4[:, :, None] * m2[None, None, :], axis=(3, 4))
        t5 = jnp.sum(m1[:, None, None, :] * m2[None, :, :, :], axis=3)
        c5 = jnp.sum(m1[:, None, None] * t5[None], axis=3)
        t6 = jnp.sum(m3[:, None, :, :, None] * m3[None, :, :, None, :], axis=2)
        c6 = jnp.sum(t6[:, :, None] * m2[None, None, :], axis=(3, 4))
        t7 = jnp.sum(m3[:, None] * m2f[None, :, :, None], axis=2)
        c7 = jnp.sum(t7[:, :, None] * m1[None, None, :], axis=3)

        rows = [m0]
        rows += [c1[i, j][None] for i, j in zip(_I2, _J2)]
        rows += [c2[i, j][None] for i, j in zip(_I2, _J2)]
        rows += [c3[i, j][None] for i, j in zip(_I2, _J2)]
        rows += [c4[i, j, k][None] for i, j, k in _TRIL3]
        for i, j in zip(_I2, _J2):
            rows.append(c5[i, j])
        for i, j in zip(_I2, _J2):
            rows.append(c6[i, j])
        rows.append(c7.reshape(125, A))
        gm = jnp.concatenate(rows, axis=0)

        h = jax.lax.dot_general(w0_ref[...], gm, (((1,), (0,)), ((), ())),
                                preferred_element_type=jnp.float32)
        h = h * (1.0 / np.sqrt(float(FEAT_DIM))) + 0.1 * b0_ref[...]
        h = _swish(h)
        h = jax.lax.dot_general(w1_ref[...], h, (((1,), (0,)), ((), ())),
                                preferred_element_type=jnp.float32)
        h = h * (1.0 / np.sqrt(512.0)) + 0.1 * b1_ref[...]
        h = _swish(h)
        o = jax.lax.dot_general(w2_ref[...], h, (((1,), (0,)), ((), ())),
                                preferred_element_type=jnp.float32)
        o = o * (1.0 / np.sqrt(512.0)) + 0.1 * b2_ref[...]
        o = scale_ref[...] * o + shift_ref[...]
        out_ref[...] = jnp.where(mask_ref[...] > 0.0, o, 0.0)

    out = pl.pallas_call(
        _k2,
        grid=(NA_PAD // A2_BLK,),
        in_specs=[
            pl.BlockSpec((200, A2_BLK), lambda i: (0, i)),
            pl.BlockSpec((512, FEAT_DIM), lambda i: (0, 0)),
            pl.BlockSpec((512, 1), lambda i: (0, 0)),
            pl.BlockSpec((512, 512), lambda i: (0, 0)),
            pl.BlockSpec((512, 1), lambda i: (0, 0)),
            pl.BlockSpec((1, 512), lambda i: (0, 0)),
            pl.BlockSpec((1, 1), lambda i: (0, 0)),
            pl.BlockSpec((1, A2_BLK), lambda i: (0, i)),
            pl.BlockSpec((1, A2_BLK), lambda i: (0, i)),
            pl.BlockSpec((1, A2_BLK), lambda i: (0, i)),
        ],
        out_specs=pl.BlockSpec((1, A2_BLK), lambda i: (0, i)),
        out_shape=jax.ShapeDtypeStruct((1, NA_PAD), jnp.float32),
    )(mT, W0.T, b0[:, None], W1.T, b1[:, None], W2.T, b2[:, None],
      scaleZ, shiftZ, maskZ)

    return out[0, :N_ATOMS][:, None]


def kernel(R, Z, neighbor_idx, radial_emb, W0, b0, W1, b1, W2, b2, scale, shift):
    return _impl(R, Z, neighbor_idx, radial_emb,
                 W0, b0, W1, b1, W2, b2, scale, shift)


# speedup vs baseline: 1.2476x; 1.2476x over previous
import functools
import numpy as np
import jax
import jax.numpy as jnp
from jax.experimental import pallas as pl

N_ATOMS = 10000
N_EDGES = 320000
N_BASIS = 7
N_RADIAL = 5
N_SPECIES = 119
R_MIN = 0.5
R_MAX = 6.0
FEAT_DIM = 360

NA_PAD = 10240
A_BLK = 2560
E_BLK = 1024
A2_BLK = 2048

_BETTA = float(N_BASIS) ** 2 / R_MAX ** 2
_RAD_NORM = (2.0 * _BETTA / np.pi) ** 0.25
_SHIFTS = np.asarray(
    R_MIN + (R_MAX - R_MIN) / N_BASIS * np.arange(N_BASIS), dtype=np.float32
)

_I2, _J2 = np.tril_indices(N_RADIAL)
_TRIL3 = [(i, j, k) for i in range(N_RADIAL)
          for j in range(i + 1) for k in range(j + 1)]


def _edge_kernel(coeffs_ref, ri_ref, rj_ref, out_ref):
    dr = ri_ref[...] - rj_ref[...]
    dr2 = jnp.sum(dr * dr, axis=0, keepdims=True)
    dist = jnp.sqrt(dr2 + 1e-12)
    e = dr / dist
    shifts = R_MIN + (R_MAX - R_MIN) / N_BASIS * jax.lax.broadcasted_iota(
        jnp.int32, (N_BASIS, 1), 0).astype(jnp.float32)
    basis = _RAD_NORM * jnp.exp(-_BETTA * (dist - shifts) ** 2)
    coeffs = coeffs_ref[...].reshape(N_RADIAL, N_BASIS, E_BLK)
    radial = jnp.sum(coeffs * basis[None, :, :], axis=1)
    cutoff = jnp.where(
        dist < R_MAX, 0.5 * (jnp.cos(np.pi / R_MAX * dist) + 1.0), 0.0
    )
    out_ref[0:N_RADIAL, :] = radial * cutoff
    out_ref[N_RADIAL:, :] = e


def _segsum_kernel(red_ref, idx_ref, m_ref):
    a = pl.program_id(0)
    e_step = pl.program_id(1)

    @pl.when(e_step == 0)
    def _():
        m_ref[...] = jnp.zeros_like(m_ref)

    radial = red_ref[0:N_RADIAL, :]
    e = red_ref[N_RADIAL:, :]
    ee = (e[:, None, :] * e[None, :, :]).reshape(9, E_BLK)
    eee = (ee[:, None, :] * e[None, :, :]).reshape(27, E_BLK)
    ones = jnp.ones((1, E_BLK), jnp.float32)
    eprod = jnp.concatenate([ones, e, ee, eee], axis=0)
    payload = (radial[:, None, :] * eprod[None, :, :]).reshape(200, E_BLK)

    row = jax.lax.broadcasted_iota(
        jnp.int32, (A_BLK, E_BLK), 0).astype(jnp.float32)
    base = (a * A_BLK).astype(jnp.float32)
    onehot = jnp.where(row + base == idx_ref[...], 1.0, 0.0)
    m_ref[...] += jax.lax.dot_general(
        onehot, payload, (((1,), (1,)), ((), ())),
        preferred_element_type=jnp.float32)


def _swish(x):
    return x / (1.0 + jnp.exp(-x))


@functools.partial(jax.jit)
def _impl(R, Z, neighbor_idx, radial_emb, W0, b0, W1, b1, W2, b2, scale, shift):
    idx_i = neighbor_idx[0].astype(jnp.int32)
    idx_j = neighbor_idx[1].astype(jnp.int32)
    Zi = Z[idx_i]
    Zj = Z[idx_j]
    riT = R[idx_i].T
    rjT = R[idx_j].T
    coeffsT = radial_emb.reshape(N_SPECIES * N_SPECIES, N_RADIAL * N_BASIS)[
        Zi * N_SPECIES + Zj].T

    red = pl.pallas_call(
        _edge_kernel,
        grid=(N_EDGES // E_BLK,),
        in_specs=[
            pl.BlockSpec((N_RADIAL * N_BASIS, E_BLK), lambda e: (0, e)),
            pl.BlockSpec((3, E_BLK), lambda e: (0, e)),
            pl.BlockSpec((3, E_BLK), lambda e: (0, e)),
        ],
        out_specs=pl.BlockSpec((8, E_BLK), lambda e: (0, e)),
        out_shape=jax.ShapeDtypeStruct((8, N_EDGES), jnp.float32),
    )(coeffsT, riT, rjT)

    idxf = idx_i.astype(jnp.float32)[None, :]
    m = pl.pallas_call(
        _segsum_kernel,
        grid=(NA_PAD // A_BLK, N_EDGES // E_BLK),
        in_specs=[
            pl.BlockSpec((8, E_BLK), lambda a, e: (0, e)),
            pl.BlockSpec((1, E_BLK), lambda a, e: (0, e)),
        ],
        out_specs=pl.BlockSpec((A_BLK, 200), lambda a, e: (a, 0)),
        out_shape=jax.ShapeDtypeStruct((NA_PAD, 200), jnp.float32),
    )(red, idxf)

    mT = m.T
    maskZ = jnp.zeros((1, NA_PAD), jnp.float32).at[0, :N_ATOMS].set(
        (Z > 0).astype(jnp.float32))
    scaleZ = jnp.zeros((1, NA_PAD), jnp.float32).at[0, :N_ATOMS].set(
        scale[Z, 0])
    shiftZ = jnp.zeros((1, NA_PAD), jnp.float32).at[0, :N_ATOMS].set(
        shift[Z, 0])

    def _k2(m_ref, w0_ref, b0_ref, w1_ref, b1_ref, w2_ref, b2_ref,
            scale_ref, shift_ref, mask_ref, out_ref):
        A = A2_BLK
        mm = m_ref[...].reshape(N_RADIAL, 40, A)
        m0 = mm[:, 0, :]
        m1 = mm[:, 1:4, :]
        m2f = mm[:, 4:13, :]
        m2 = m2f.reshape(N_RADIAL, 3, 3, A)
        m3f = mm[:, 13:40, :]
        m3 = m3f.reshape(N_RADIAL, 9, 3, A)

        c1 = jnp.sum(m1[:, None] * m1[None, :], axis=2)
        c2 = jnp.sum(m2f[:, None] * m2f[None, :], axis=2)
        c3 = jnp.sum(m3f[:, None] * m3f[None, :], axis=2)
        t4 = jnp.sum(m2[:, None, :, :, None] * m2[None, :, :, None, :], axis=2)
        c4 = jnp.sum(t4[:, :, None] * m2[None, None, :], axis=(3, 4))
        t5 = jnp.sum(m1[:, None, None, :] * m2[None, :, :, :], axis=3)
        c5 = jnp.sum(m1[:, None, None] * t5[None], axis=3)
        t6 = jnp.sum(m3[:, None, :, :, None] * m3[None, :, :, None, :], axis=2)
        c6 = jnp.sum(t6[:, :, None] * m2[None, None, :], axis=(3, 4))
        t7 = jnp.sum(m3[:, None] * m2f[None, :, :, None], axis=2)
        c7 = jnp.sum(t7[:, :, None] * m1[None, None, :], axis=3)

        rows = [m0]
        rows += [c1[i, j][None] for i, j in zip(_I2, _J2)]
        rows += [c2[i, j][None] for i, j in zip(_I2, _J2)]
        rows += [c3[i, j][None] for i, j in zip(_I2, _J2)]
        rows += [c4[i, j, k][None] for i, j, k in _TRIL3]
        for i, j in zip(_I2, _J2):
            rows.append(c5[i, j])
        for i, j in zip(_I2, _J2):
            rows.append(c6[i, j])
        rows.append(c7.reshape(125, A))
        gm = jnp.concatenate(rows, axis=0)

        h = jax.lax.dot_general(w0_ref[...], gm, (((1,), (0,)), ((), ())),
                                preferred_element_type=jnp.float32)
        h = h * (1.0 / np.sqrt(float(FEAT_DIM))) + 0.1 * b0_ref[...]
        h = _swish(h)
        h = jax.lax.dot_general(w1_ref[...], h, (((1,), (0,)), ((), ())),
                                preferred_element_type=jnp.float32)
        h = h * (1.0 / np.sqrt(512.0)) + 0.1 * b1_ref[...]
        h = _swish(h)
        o = jax.lax.dot_general(w2_ref[...], h, (((1,), (0,)), ((), ())),
                                preferred_element_type=jnp.float32)
        o = o * (1.0 / np.sqrt(512.0)) + 0.1 * b2_ref[...]
        o = scale_ref[...] * o + shift_ref[...]
        out_ref[...] = jnp.where(mask_ref[...] > 0.0, o, 0.0)

    out = pl.pallas_call(
        _k2,
        grid=(NA_PAD // A2_BLK,),
        in_specs=[
            pl.BlockSpec((200, A2_BLK), lambda i: (0, i)),
            pl.BlockSpec((512, FEAT_DIM), lambda i: (0, 0)),
            pl.BlockSpec((512, 1), lambda i: (0, 0)),
            pl.BlockSpec((512, 512), lambda i: (0, 0)),
            pl.BlockSpec((512, 1), lambda i: (0, 0)),
            pl.BlockSpec((1, 512), lambda i: (0, 0)),
            pl.BlockSpec((1, 1), lambda i: (0, 0)),
            pl.BlockSpec((1, A2_BLK), lambda i: (0, i)),
            pl.BlockSpec((1, A2_BLK), lambda i: (0, i)),
            pl.BlockSpec((1, A2_BLK), lambda i: (0, i)),
        ],
        out_specs=pl.BlockSpec((1, A2_BLK), lambda i: (0, i)),
        out_shape=jax.ShapeDtypeStruct((1, NA_PAD), jnp.float32),
    )(mT, W0.T, b0[:, None], W1.T, b1[:, None], W2.T, b2[:, None],
      scaleZ, shiftZ, maskZ)

    return out[0, :N_ATOMS][:, None]


def kernel(R, Z, neighbor_idx, radial_emb, W0, b0, W1, b1, W2, b2, scale, shift):
    return _impl(R, Z, neighbor_idx, radial_emb,
                 W0, b0, W1, b1, W2, b2, scale, shift)
